# baseline (speedup 1.0000x reference)
import functools

import jax
import jax.numpy as jnp
from jax import lax
from jax.experimental import pallas as pl
from jax.experimental.pallas import tpu as pltpu
from jax.experimental.pallas import tpu_sc as plsc

_C = 10
_E = 64
_A = 32


def _sc_gather(user_ids, item_ids, user_table, item_table):
    B = user_ids.shape[0]
    E = user_table.shape[1]
    info = plsc.get_sparse_core_info()
    nw = info.num_cores * info.num_subcores
    bpw = B // nw
    ch = 128
    nch = bpw // ch

    mesh = plsc.VectorSubcoreMesh(core_axis_name="c", subcore_axis_name="s")

    @functools.partial(
        pl.kernel,
        mesh=mesh,
        compiler_params=pltpu.CompilerParams(use_tc_tiling_on_sc=False),
        out_type=(
            jax.ShapeDtypeStruct((B, E), jnp.float32),
            jax.ShapeDtypeStruct((B, E), jnp.float32),
        ),
        scratch_types=[
            pltpu.VMEM((bpw,), jnp.int32),
            pltpu.VMEM((bpw,), jnp.int32),
            pltpu.VMEM((bpw, E), jnp.float32),
            pltpu.VMEM((bpw, E), jnp.float32),
            pltpu.SemaphoreType.DMA,
        ],
    )
    def body(uid_hbm, iid_hbm, utab_hbm, itab_hbm, out_u, out_i,
             idx_u, idx_i, rows_u, rows_i, sem):
        wid = lax.axis_index("s") * info.num_cores + lax.axis_index("c")
        base = wid * bpw
        pltpu.sync_copy(uid_hbm.at[pl.ds(base, bpw)], idx_u)
        pltpu.sync_copy(iid_hbm.at[pl.ds(base, bpw)], idx_i)
        descs = []
        for c in range(nch):
            sl = pl.ds(c * ch, ch)
            descs.append(
                pltpu.async_copy(utab_hbm.at[idx_u.at[sl]], rows_u.at[sl], sem))
            descs.append(
                pltpu.async_copy(itab_hbm.at[idx_i.at[sl]], rows_i.at[sl], sem))
        for d in descs:
            d.wait()
        pltpu.sync_copy(rows_u, out_u.at[pl.ds(base, bpw)])
        pltpu.sync_copy(rows_i, out_i.at[pl.ds(base, bpw)])

    return body(user_ids, item_ids, user_table, item_table)


def _tc_attend(ids, u_rows, i_rows, ct, W, b_row, v):
    B = u_rows.shape[0]
    bm = 4096
    nb = B // bm

    def tc_body(ids_ref, u_ref, i_ref, ct_ref, w_ref, b_ref, v_ref, out_ref):
        ct_full = ct_ref[...]
        q = jnp.tanh(
            jnp.dot(ct_full, w_ref[...],
                    preferred_element_type=jnp.float32) + b_ref[...])
        logit = jnp.dot(q, v_ref[...],
                        preferred_element_type=jnp.float32)
        e = jnp.exp(logit - jnp.max(logit, axis=0, keepdims=True))
        idv = ids_ref[...]
        den = jnp.zeros((bm, 1), jnp.float32)
        ca_num = jnp.zeros((bm, _E), jnp.float32)
        for g in range(_C):
            n_g = jnp.sum((idv == g).astype(jnp.float32), axis=1,
                          keepdims=True)
            e_g = lax.slice(e, (g, 0), (g + 1, 1))
            w_g = n_g * e_g
            den = den + w_g
            ca_num = ca_num + w_g * lax.slice(ct_full, (g, 0), (g + 1, _E))
        ca = ca_num / den
        s = jnp.sum(u_ref[...] * (i_ref[...] + ca), axis=1, keepdims=True)
        out_ref[...] = s

    out = pl.pallas_call(
        tc_body,
        grid=(nb,),
        in_specs=[
            pl.BlockSpec((bm, _C), lambda i: (i, 0)),
            pl.BlockSpec((bm, _E), lambda i: (i, 0)),
            pl.BlockSpec((bm, _E), lambda i: (i, 0)),
            pl.BlockSpec((_C, _E), lambda i: (0, 0)),
            pl.BlockSpec((_E, _A), lambda i: (0, 0)),
            pl.BlockSpec((1, _A), lambda i: (0, 0)),
            pl.BlockSpec((_A, 1), lambda i: (0, 0)),
        ],
        out_specs=pl.BlockSpec((bm, 1), lambda i: (i, 0)),
        out_shape=jax.ShapeDtypeStruct((B, 1), jnp.float32),
    )(ids, u_rows, i_rows, ct, W, b_row, v)
    return out[:, 0]


def kernel(user_ids, item_ids, component_ids, user_table, item_table,
           component_table, W, b, v):
    u_rows, i_rows = _sc_gather(user_ids, item_ids, user_table, item_table)
    return _tc_attend(component_ids, u_rows, i_rows, component_table, W,
                      b.reshape(1, _A), v)
